# baseline (speedup 1.0000x reference)
import functools

import jax
import jax.numpy as jnp
from jax import lax
from jax.experimental import pallas as pl
from jax.experimental.pallas import tpu as pltpu
from jax.experimental.pallas import tpu_sc as plsc

NUM_KNOTS = 10
LOG_DEPTH_MIN = -3.0
LOG_DEPTH_MAX = 5.0
REF_IDX = 3
INV_H = (NUM_KNOTS - 1) / (LOG_DEPTH_MAX - LOG_DEPTH_MIN)

NC, NS, L = 2, 16, 16
NW = NC * NS

N_TOTAL = 16 * 512 * 512
PER_W = N_TOTAL // NW
CHUNK = 16384
NCHUNK = PER_W // CHUNK
UNROLL = 32


def _spline_body(ld_hbm, ky_hbm, out_hbm,
                 ky_v, y2_v, dy_v, inb, outb,
                 sem_ky, sem_in0, sem_in1, sem_out0, sem_out1):
    wid = lax.axis_index("s") * NC + lax.axis_index("c")
    base = wid * PER_W

    sem_in = (sem_in0, sem_in1)
    sem_out = (sem_out0, sem_out1)

    def start_in(i, b):
        return pltpu.async_copy(
            ld_hbm.at[pl.ds(base + i * CHUNK, CHUNK)], inb.at[b], sem_in[b])

    def start_out(i, b):
        return pltpu.async_copy(
            outb.at[b], out_hbm.at[pl.ds(base + i * CHUNK, CHUNK)], sem_out[b])

    h_in0 = start_in(0, 0)
    h_in1 = start_in(1, 1)

    pltpu.sync_copy(ky_hbm, ky_v.at[pl.ds(0, NUM_KNOTS)])
    ky = ky_v[...]
    lane = lax.iota(jnp.int32, 16)
    y2 = jnp.where(lane == REF_IDX, 0.0, ky)
    y2_v[...] = y2
    y2_next = plsc.load_gather(y2_v, [jnp.minimum(lane + 1, 15)])
    dy = jnp.maximum(y2_next - y2, 0.0)
    dy_v[...] = dy

    u_hi = jnp.float32(8.999999)

    def compute(b):
        @plsc.parallel_loop(0, CHUNK, step=L, unroll=UNROLL)
        def body(off):
            v = inb[b, pl.ds(off, L)]
            u = v * INV_H + (-LOG_DEPTH_MIN * INV_H)
            u = jnp.minimum(jnp.maximum(u, 0.0), u_hi)
            idx = u.astype(jnp.int32)
            t = u - idx.astype(jnp.float32)
            y0 = plsc.load_gather(y2_v, [idx])
            dyi = plsc.load_gather(dy_v, [idx])
            res = y0 + t * dyi
            oob = (v < LOG_DEPTH_MIN) | (v > LOG_DEPTH_MAX)
            outb[b, pl.ds(off, L)] = jnp.where(oob, v, res)

    h_in = [None] * NCHUNK
    h_out = [None] * NCHUNK
    h_in[0], h_in[1] = h_in0, h_in1
    for i in range(NCHUNK):
        b = i % 2
        h_in[i].wait()
        if i >= 2:
            h_out[i - 2].wait()
        compute(b)
        h_out[i] = start_out(i, b)
        if i + 2 < NCHUNK:
            h_in[i + 2] = start_in(i + 2, b)
    h_out[NCHUNK - 2].wait()
    h_out[NCHUNK - 1].wait()


@functools.partial(jax.jit, static_argnames=())
def kernel(log_depth, knots_y):
    ld_flat = log_depth.reshape(-1)
    mesh = plsc.VectorSubcoreMesh(core_axis_name="c", subcore_axis_name="s")
    run = functools.partial(
        pl.kernel,
        out_type=jax.ShapeDtypeStruct((N_TOTAL,), jnp.float32),
        mesh=mesh,
        compiler_params=pltpu.CompilerParams(needs_layout_passes=False),
        scratch_types=[
            pltpu.VMEM((16,), jnp.float32),
            pltpu.VMEM((16,), jnp.float32),
            pltpu.VMEM((16,), jnp.float32),
            pltpu.VMEM((2, CHUNK), jnp.float32),
            pltpu.VMEM((2, CHUNK), jnp.float32),
            pltpu.SemaphoreType.DMA,
            pltpu.SemaphoreType.DMA,
            pltpu.SemaphoreType.DMA,
            pltpu.SemaphoreType.DMA,
            pltpu.SemaphoreType.DMA,
        ],
    )(_spline_body)
    out = run(ld_flat, knots_y)
    return out.reshape(log_depth.shape)


# speedup vs baseline: 2.0558x; 2.0558x over previous
import functools

import jax
import jax.numpy as jnp
from jax import lax
from jax.experimental import pallas as pl
from jax.experimental.pallas import tpu as pltpu
from jax.experimental.pallas import tpu_sc as plsc

NUM_KNOTS = 10
LOG_DEPTH_MIN = -3.0
LOG_DEPTH_MAX = 5.0
REF_IDX = 3
INV_H = (NUM_KNOTS - 1) / (LOG_DEPTH_MAX - LOG_DEPTH_MIN)

NC, NS, L = 2, 16, 16
NW = NC * NS

N_TOTAL = 16 * 512 * 512
PER_W = N_TOTAL // NW
CHUNK = 16384
NCHUNK = PER_W // CHUNK
UNROLL = 16


def _spline_body(ld_hbm, ky_hbm, out_hbm,
                 ky_v, y2_v, dy_v, b_v, inb, outb,
                 sem_ky, sem_in0, sem_in1, sem_out0, sem_out1):
    wid = lax.axis_index("s") * NC + lax.axis_index("c")
    base = wid * PER_W

    sem_in = (sem_in0, sem_in1)
    sem_out = (sem_out0, sem_out1)

    def start_in(i, b):
        return pltpu.async_copy(
            ld_hbm.at[pl.ds(base + i * CHUNK, CHUNK)], inb.at[b], sem_in[b])

    def start_out(i, b):
        return pltpu.async_copy(
            outb.at[b], out_hbm.at[pl.ds(base + i * CHUNK, CHUNK)], sem_out[b])

    h_in0 = start_in(0, 0)
    h_in1 = start_in(1, 1)

    pltpu.sync_copy(ky_hbm, ky_v.at[pl.ds(0, NUM_KNOTS)])
    ky = ky_v[...]
    lane = lax.iota(jnp.int32, 16)
    y2 = jnp.where(lane == REF_IDX, 0.0, ky)
    y2_v[...] = y2
    y2_next = plsc.load_gather(y2_v, [jnp.minimum(lane + 1, 15)])
    dy = jnp.maximum(y2_next - y2, 0.0)
    dy_v[...] = dy
    b_v[...] = y2 - lane.astype(jnp.float32) * dy

    u_hi = jnp.float32(8.999999)

    def compute(b):
        @plsc.parallel_loop(0, CHUNK, step=L, unroll=UNROLL)
        def body(off):
            v = inb[b, pl.ds(off, L)]
            u = v * INV_H + (-LOG_DEPTH_MIN * INV_H)
            u = jnp.minimum(jnp.maximum(u, 0.0), u_hi)
            idx = u.astype(jnp.int32)
            bi = plsc.load_gather(b_v, [idx])
            ai = plsc.load_gather(dy_v, [idx])
            res = bi + u * ai
            oob = (v < LOG_DEPTH_MIN) | (v > LOG_DEPTH_MAX)
            outb[b, pl.ds(off, L)] = jnp.where(oob, v, res)

    h_in = [None] * NCHUNK
    h_out = [None] * NCHUNK
    h_in[0], h_in[1] = h_in0, h_in1
    for i in range(NCHUNK):
        b = i % 2
        h_in[i].wait()
        if i >= 2:
            h_out[i - 2].wait()
        compute(b)
        h_out[i] = start_out(i, b)
        if i + 2 < NCHUNK:
            h_in[i + 2] = start_in(i + 2, b)
    h_out[NCHUNK - 2].wait()
    h_out[NCHUNK - 1].wait()


@functools.partial(jax.jit, static_argnames=())
def kernel(log_depth, knots_y):
    ld_flat = log_depth.reshape(-1)
    mesh = plsc.VectorSubcoreMesh(core_axis_name="c", subcore_axis_name="s")
    run = functools.partial(
        pl.kernel,
        out_type=jax.ShapeDtypeStruct((N_TOTAL,), jnp.float32),
        mesh=mesh,
        compiler_params=pltpu.CompilerParams(needs_layout_passes=False),
        scratch_types=[
            pltpu.VMEM((16,), jnp.float32),
            pltpu.VMEM((16,), jnp.float32),
            pltpu.VMEM((16,), jnp.float32),
            pltpu.VMEM((16,), jnp.float32),
            pltpu.VMEM((2, CHUNK), jnp.float32),
            pltpu.VMEM((2, CHUNK), jnp.float32),
            pltpu.SemaphoreType.DMA,
            pltpu.SemaphoreType.DMA,
            pltpu.SemaphoreType.DMA,
            pltpu.SemaphoreType.DMA,
            pltpu.SemaphoreType.DMA,
        ],
    )(_spline_body)
    out = run(ld_flat, knots_y)
    return out.reshape(log_depth.shape)
